# baseline (speedup 1.0000x reference)
import functools

import jax
import jax.numpy as jnp
from jax import lax
from jax.experimental import pallas as pl
from jax.experimental.pallas import tpu as pltpu
from jax.experimental.pallas import tpu_sc as plsc

N = 10000
E = 320000
D = 128
NPAD = 10240
ROWS = 512

NTILES = 32
RPT = NPAD // NTILES
NR = RPT + 8
EC = 6400
NCHUNK = E // EC
GR = 128
CAP = 14000
CAPG = 14080
SB = 14


def _dense_body(x_ref, nb_ref, wfx_ref, wfn_ref, bf_ref, wp_ref, bp_ref, o_ref, *, relu):
    x = x_ref[...]
    nb = nb_ref[...]
    h = (jnp.dot(x, wfx_ref[...], preferred_element_type=jnp.float32)
         + jnp.dot(nb, wfn_ref[...], preferred_element_type=jnp.float32)
         + bf_ref[...])
    if relu:
        h = jnp.maximum(h, 0.0)
    o_ref[...] = (jnp.dot(x, wp_ref[...], preferred_element_type=jnp.float32)
                  + bp_ref[...] + h)


def _dense(x, neigh, W_fc, b_fc, W_proj, b_proj, relu):
    Do = W_fc.shape[0]
    if Do < 128:
        W_fc = jnp.pad(W_fc, ((0, 128 - Do), (0, 0)))
        b_fc = jnp.pad(b_fc, (0, 128 - Do))
        W_proj = jnp.pad(W_proj, ((0, 128 - Do), (0, 0)))
        b_proj = jnp.pad(b_proj, (0, 128 - Do))
        Do = 128
    wfx = W_fc[:, :D].T
    wfn = W_fc[:, D:].T
    wp = W_proj.T
    bf = b_fc[None, :]
    bp = b_proj[None, :]
    grid = NPAD // ROWS
    return pl.pallas_call(
        functools.partial(_dense_body, relu=relu),
        grid=(grid,),
        in_specs=[
            pl.BlockSpec((ROWS, D), lambda i: (i, 0)),
            pl.BlockSpec((ROWS, D), lambda i: (i, 0)),
            pl.BlockSpec((D, Do), lambda i: (0, 0)),
            pl.BlockSpec((D, Do), lambda i: (0, 0)),
            pl.BlockSpec((1, Do), lambda i: (0, 0)),
            pl.BlockSpec((D, Do), lambda i: (0, 0)),
            pl.BlockSpec((1, Do), lambda i: (0, 0)),
        ],
        out_specs=pl.BlockSpec((ROWS, Do), lambda i: (i, 0)),
        out_shape=jax.ShapeDtypeStruct((NPAD, Do), jnp.float32),
    )(x, neigh, wfx, wfn, bf, wp, bp)


def _seg_max_sc(x, packed):
    mesh = plsc.VectorSubcoreMesh(core_axis_name="c", subcore_axis_name="s")

    @functools.partial(
        pl.kernel, mesh=mesh,
        out_type=jax.ShapeDtypeStruct((NPAD, D), jnp.float32),
        scratch_types=[
            pltpu.VMEM((EC,), jnp.int32),
            pltpu.VMEM((CAP,), jnp.int32),
            pltpu.VMEM((CAP,), jnp.int32),
            pltpu.VMEM((CAPG,), jnp.int32),
            pltpu.VMEM((352,), jnp.int32),
            pltpu.VMEM((352,), jnp.int32),
            pltpu.VMEM((352,), jnp.int32),
            pltpu.VMEM((2 * GR, D), jnp.float32),
            pltpu.VMEM((NR, D), jnp.float32),
            pltpu.SemaphoreType.DMA,
            pltpu.SemaphoreType.DMA,
        ],
        compiler_params=pltpu.CompilerParams(needs_layout_passes=False),
    )
    def k(x_hbm, pe_hbm, out_hbm, pbuf, msrc, mdst, ssrc,
          counts, starts, fill, rows, acc, sem0, sem1):
        wid = lax.axis_index("s") * 2 + lax.axis_index("c")
        lo = wid * RPT

        neg16 = jnp.full((16,), -jnp.inf, jnp.float32)
        zero16i = jnp.zeros((16,), jnp.int32)
        dummy16i = jnp.full((16,), RPT, jnp.int32)

        @plsc.parallel_loop(0, NR, unroll=4)
        def _init_acc(r):
            for v in range(8):
                acc[r, pl.ds(v * 16, 16)] = neg16

        @plsc.parallel_loop(0, CAP // 16, unroll=4)
        def _init_m(i):
            msrc[pl.ds(i * 16, 16)] = zero16i
            mdst[pl.ds(i * 16, 16)] = dummy16i

        @plsc.parallel_loop(0, CAPG // 16, unroll=4)
        def _init_s(i):
            ssrc[pl.ds(i * 16, 16)] = zero16i

        lo16k = lo * (1 << SB)
        hi16k = (lo + RPT) * (1 << SB)
        mask_s = (1 << SB) - 1

        def round_body(c0):
            def scan_cond(st):
                c, cur = st
                return (c < NCHUNK) & (cur <= CAP - EC)

            def scan_chunk(st):
                c, cur = st
                pltpu.sync_copy(pe_hbm.at[pl.ds(c * EC, EC)], pbuf)

                @plsc.parallel_loop(0, EC // 16, unroll=4, carry=cur)
                def scan_step(i, cur_):
                    p = pbuf[pl.ds(i * 16, 16)]
                    m = (p >= lo16k) & (p < hi16k)
                    pos = plsc.cumsum(m.astype(jnp.int32))
                    off16 = cur_ + pos - 1
                    plsc.store_scatter(msrc, [off16], p & mask_s, mask=m)
                    plsc.store_scatter(mdst, [off16],
                                       lax.shift_right_logical(p, SB) - lo, mask=m)
                    return cur_ + pos[15]
                return c + 1, scan_step

            c, cur = lax.while_loop(scan_cond, scan_chunk, (c0, 0))
            nv = lax.shift_right_logical(cur + 15, 4)

            @plsc.parallel_loop(0, 352 // 16, unroll=2)
            def _init_c(i):
                counts[pl.ds(i * 16, 16)] = zero16i

            @plsc.parallel_loop(0, nv, unroll=4)
            def _hist_step(i):
                dv = mdst[pl.ds(i * 16, 16)]
                rk, lastm = plsc.scan_count(dv)
                plsc.addupdate_scatter(counts, [dv], rk, mask=lastm)

            def prefix_step(i, base):
                cv = counts[pl.ds(i * 16, 16)]
                pos = plsc.cumsum(cv)
                starts[pl.ds(i * 16, 16)] = base + pos - cv
                fill[pl.ds(i * 16, 16)] = base + pos - cv
                return base + pos[15]
            lax.fori_loop(0, 352 // 16, prefix_step, 0)

            def place_step(i, carry):
                sl = pl.ds(i * 16, 16)
                dv = mdst[sl]
                sv = msrc[sl]
                rk, lastm = plsc.scan_count(dv)
                base = plsc.load_gather(fill, [dv])
                plsc.store_scatter(ssrc, [base + rk - 1], sv)
                plsc.addupdate_scatter(fill, [dv], rk, mask=lastm)
                return carry
            lax.fori_loop(0, nv, place_step, 0)

            ngr = lax.shift_right_logical(nv * 16 + (GR - 1), 7)

            def fire(g, half, sem):
                @pl.when(g < ngr)
                def _():
                    pltpu.async_copy(x_hbm.at[ssrc.at[pl.ds(g * GR, GR)]],
                                     rows.at[pl.ds(half * GR, GR)], sem)

            def wait(half, sem):
                pltpu.make_async_copy(x_hbm.at[ssrc.at[pl.ds(0, GR)]],
                                      rows.at[pl.ds(half * GR, GR)], sem).wait()

            def rmw_half(g, half, rlo):
                gend = g * GR + GR
                gbase = g * GR - half * GR

                def row_cond(st):
                    r, _ = st
                    return (starts[pl.ds(r, 16)][0] < gend) & (r < NR - 1)

                def row_step(st):
                    r, _ = st
                    svec = starts[pl.ds(r, 16)]
                    s0 = jnp.maximum(svec[0], g * GR)
                    s1 = jnp.minimum(svec[1], gend)
                    a = tuple(acc[r, pl.ds(v * 16, 16)] for v in range(8))

                    def s_step(s, a_):
                        return tuple(
                            jnp.maximum(a_[v], rows[s - gbase, pl.ds(v * 16, 16)])
                            for v in range(8))
                    a = lax.fori_loop(s0, s1, s_step, a)
                    for v in range(8):
                        acc[r, pl.ds(v * 16, 16)] = a[v]
                    return r + 1, 0

                rf, _ = lax.while_loop(row_cond, row_step, (rlo, 0))
                return jnp.maximum(rf - 1, 0)

            fire(0, 0, sem0)
            fire(1, 1, sem1)

            def pair_step(q, rlo):
                g = 2 * q

                def do_even(rl):
                    wait(0, sem0)
                    r = rmw_half(g, 0, rl)
                    fire(g + 2, 0, sem0)
                    return r
                r1 = lax.cond(g < ngr, do_even, lambda rl: rl, rlo)

                def do_odd(rl):
                    wait(1, sem1)
                    r = rmw_half(g + 1, 1, rl)
                    fire(g + 3, 1, sem1)
                    return r
                r2 = lax.cond(g + 1 < ngr, do_odd, lambda rl: rl, r1)
                return r2
            lax.fori_loop(0, lax.shift_right_logical(ngr + 1, 1), pair_step, 0)
            return c

        lax.while_loop(lambda c_: c_ < NCHUNK, round_body, 0)

        zero16 = jnp.zeros((16,), jnp.float32)

        @plsc.parallel_loop(0, RPT, unroll=4)
        def _fix_r(r):
            for v in range(8):
                sl = pl.ds(v * 16, 16)
                a = acc[r, sl]
                acc[r, sl] = jnp.where(a == neg16, zero16, a)
        pltpu.sync_copy(acc.at[pl.ds(0, RPT)], out_hbm.at[pl.ds(lo, RPT)])

    return k(x, packed)


def kernel(x, edge_index0, edge_index1, edge_index2,
           W_fc1, b_fc1, W_proj1, b_proj1,
           W_fc2, b_fc2, W_proj2, b_proj2,
           W_fc3, b_fc3, W_proj3, b_proj3):
    pe0 = (edge_index0[1] << SB) | edge_index0[0]
    pe1 = (edge_index1[1] << SB) | edge_index1[0]
    pe2 = (edge_index2[1] << SB) | edge_index2[0]

    n1 = _seg_max_sc(x, pe0)
    xp = jnp.pad(x, ((0, NPAD - N), (0, 0)))
    h1 = _dense(xp, n1, W_fc1, b_fc1, W_proj1, b_proj1, relu=True)

    n2 = _seg_max_sc(h1, pe1)
    h2 = _dense(h1, n2, W_fc2, b_fc2, W_proj2, b_proj2, relu=False)

    n3 = _seg_max_sc(h2, pe2)
    h3 = _dense(h2, n3, W_fc3, b_fc3, W_proj3, b_proj3, relu=False)
    return h3[:N, :1]


# speedup vs baseline: 1.1229x; 1.1229x over previous
import functools

import jax
import jax.numpy as jnp
from jax import lax
from jax.experimental import pallas as pl
from jax.experimental.pallas import tpu as pltpu
from jax.experimental.pallas import tpu_sc as plsc

N = 10000
E = 320000
D = 128
NPAD = 10240
ROWS = 512

NTILES = 32
RPT = NPAD // NTILES
NR = RPT + 8
EC = 6400
NCHUNK = E // EC
GR = 128
CAP = 14000
CAPG = 14080
SB = 14


def _dense_body(x_ref, nb_ref, wfx_ref, wfn_ref, bf_ref, wp_ref, bp_ref, o_ref, *, relu):
    x = x_ref[...]
    nb = nb_ref[...]
    h = (jnp.dot(x, wfx_ref[...], preferred_element_type=jnp.float32)
         + jnp.dot(nb, wfn_ref[...], preferred_element_type=jnp.float32)
         + bf_ref[...])
    if relu:
        h = jnp.maximum(h, 0.0)
    o_ref[...] = (jnp.dot(x, wp_ref[...], preferred_element_type=jnp.float32)
                  + bp_ref[...] + h)


def _dense(x, neigh, W_fc, b_fc, W_proj, b_proj, relu):
    Do = W_fc.shape[0]
    if Do < 128:
        W_fc = jnp.pad(W_fc, ((0, 128 - Do), (0, 0)))
        b_fc = jnp.pad(b_fc, (0, 128 - Do))
        W_proj = jnp.pad(W_proj, ((0, 128 - Do), (0, 0)))
        b_proj = jnp.pad(b_proj, (0, 128 - Do))
        Do = 128
    wfx = W_fc[:, :D].T
    wfn = W_fc[:, D:].T
    wp = W_proj.T
    bf = b_fc[None, :]
    bp = b_proj[None, :]
    grid = NPAD // ROWS
    return pl.pallas_call(
        functools.partial(_dense_body, relu=relu),
        grid=(grid,),
        in_specs=[
            pl.BlockSpec((ROWS, D), lambda i: (i, 0)),
            pl.BlockSpec((ROWS, D), lambda i: (i, 0)),
            pl.BlockSpec((D, Do), lambda i: (0, 0)),
            pl.BlockSpec((D, Do), lambda i: (0, 0)),
            pl.BlockSpec((1, Do), lambda i: (0, 0)),
            pl.BlockSpec((D, Do), lambda i: (0, 0)),
            pl.BlockSpec((1, Do), lambda i: (0, 0)),
        ],
        out_specs=pl.BlockSpec((ROWS, Do), lambda i: (i, 0)),
        out_shape=jax.ShapeDtypeStruct((NPAD, Do), jnp.float32),
    )(x, neigh, wfx, wfn, bf, wp, bp)


def _seg_max_sc(x, packed):
    mesh = plsc.VectorSubcoreMesh(core_axis_name="c", subcore_axis_name="s")

    @functools.partial(
        pl.kernel, mesh=mesh,
        out_type=jax.ShapeDtypeStruct((NPAD, D), jnp.float32),
        scratch_types=[
            pltpu.VMEM((2 * EC,), jnp.int32),
            pltpu.VMEM((CAP,), jnp.int32),
            pltpu.VMEM((CAP,), jnp.int32),
            pltpu.VMEM((CAPG,), jnp.int32),
            pltpu.VMEM((352,), jnp.int32),
            pltpu.VMEM((352,), jnp.int32),
            pltpu.VMEM((352,), jnp.int32),
            pltpu.VMEM((2 * GR, D), jnp.float32),
            pltpu.VMEM((NR, D), jnp.float32),
            pltpu.SemaphoreType.DMA,
            pltpu.SemaphoreType.DMA,
            pltpu.SemaphoreType.DMA,
        ],
        compiler_params=pltpu.CompilerParams(needs_layout_passes=False),
    )
    def k(x_hbm, pe_hbm, out_hbm, pbuf, msrc, mdst, ssrc,
          counts, starts, fill, rows, acc, sem0, sem1, sem2):
        wid = lax.axis_index("s") * 2 + lax.axis_index("c")
        lo = wid * RPT

        neg16 = jnp.full((16,), -jnp.inf, jnp.float32)
        zero16i = jnp.zeros((16,), jnp.int32)
        dummy16i = jnp.full((16,), RPT, jnp.int32)

        @plsc.parallel_loop(0, NR, unroll=4)
        def _init_acc(r):
            for v in range(8):
                acc[r, pl.ds(v * 16, 16)] = neg16

        @plsc.parallel_loop(0, CAP // 16, unroll=4)
        def _init_m(i):
            msrc[pl.ds(i * 16, 16)] = zero16i
            mdst[pl.ds(i * 16, 16)] = dummy16i

        @plsc.parallel_loop(0, CAPG // 16, unroll=4)
        def _init_s(i):
            ssrc[pl.ds(i * 16, 16)] = zero16i

        lo16k = lo * (1 << SB)
        hi16k = (lo + RPT) * (1 << SB)
        mask_s = (1 << SB) - 1

        def fire_chunk(c):
            @pl.when(c < NCHUNK)
            def _():
                pltpu.async_copy(pe_hbm.at[pl.ds(c * EC, EC)],
                                 pbuf.at[pl.ds((c & 1) * EC, EC)], sem2)

        def wait_chunk(c):
            pltpu.make_async_copy(pe_hbm.at[pl.ds(0, EC)],
                                  pbuf.at[pl.ds((c & 1) * EC, EC)], sem2).wait()

        fire_chunk(0)

        def round_body(c0):
            def scan_cond(st):
                c, cur = st
                return (c < NCHUNK) & (cur <= CAP - EC)

            def scan_chunk(st):
                c, cur = st
                wait_chunk(c)
                fire_chunk(c + 1)
                pbase = (c & 1) * EC

                @plsc.parallel_loop(0, EC // 16, unroll=4, carry=cur)
                def scan_step(i, cur_):
                    p = pbuf[pl.ds(pbase + i * 16, 16)]
                    m = (p >= lo16k) & (p < hi16k)
                    pos = plsc.cumsum(m.astype(jnp.int32))
                    off16 = cur_ + pos - 1
                    plsc.store_scatter(msrc, [off16], p & mask_s, mask=m)
                    plsc.store_scatter(mdst, [off16],
                                       lax.shift_right_logical(p, SB) - lo, mask=m)
                    return cur_ + pos[15]
                return c + 1, scan_step

            c, cur = lax.while_loop(scan_cond, scan_chunk, (c0, 0))
            nv = lax.shift_right_logical(cur + 15, 4)

            @plsc.parallel_loop(0, 352 // 16, unroll=2)
            def _init_c(i):
                counts[pl.ds(i * 16, 16)] = zero16i

            @plsc.parallel_loop(0, nv, unroll=4)
            def _hist_step(i):
                dv = mdst[pl.ds(i * 16, 16)]
                rk, lastm = plsc.scan_count(dv)
                plsc.addupdate_scatter(counts, [dv], rk, mask=lastm)

            def prefix_step(i, base):
                cv = counts[pl.ds(i * 16, 16)]
                pos = plsc.cumsum(cv)
                starts[pl.ds(i * 16, 16)] = base + pos - cv
                fill[pl.ds(i * 16, 16)] = base + pos - cv
                return base + pos[15]
            lax.fori_loop(0, 352 // 16, prefix_step, 0)

            def place_step(i, carry):
                sl = pl.ds(i * 16, 16)
                dv = mdst[sl]
                sv = msrc[sl]
                rk, lastm = plsc.scan_count(dv)
                base = plsc.load_gather(fill, [dv])
                plsc.store_scatter(ssrc, [base + rk - 1], sv)
                plsc.addupdate_scatter(fill, [dv], rk, mask=lastm)
                return carry
            lax.fori_loop(0, nv, place_step, 0)

            ngr = lax.shift_right_logical(nv * 16 + (GR - 1), 7)

            def fire(g, half, sem):
                @pl.when(g < ngr)
                def _():
                    pltpu.async_copy(x_hbm.at[ssrc.at[pl.ds(g * GR, GR)]],
                                     rows.at[pl.ds(half * GR, GR)], sem)

            def wait(half, sem):
                pltpu.make_async_copy(x_hbm.at[ssrc.at[pl.ds(0, GR)]],
                                      rows.at[pl.ds(half * GR, GR)], sem).wait()

            def rmw_half(g, half, rlo):
                gend = g * GR + GR
                gbase = g * GR - half * GR

                def row_cond(st):
                    r, _ = st
                    return (starts[pl.ds(r, 16)][0] < gend) & (r < NR - 1)

                def row_step(st):
                    r, _ = st
                    svec = starts[pl.ds(r, 16)]
                    s0 = jnp.maximum(svec[0], g * GR)
                    s1 = jnp.minimum(svec[1], gend)
                    a = tuple(acc[r, pl.ds(v * 16, 16)] for v in range(8))

                    def s_step(s, a_):
                        return tuple(
                            jnp.maximum(a_[v], rows[s - gbase, pl.ds(v * 16, 16)])
                            for v in range(8))
                    a = lax.fori_loop(s0, s1, s_step, a)
                    for v in range(8):
                        acc[r, pl.ds(v * 16, 16)] = a[v]
                    return r + 1, 0

                rf, _ = lax.while_loop(row_cond, row_step, (rlo, 0))
                return jnp.maximum(rf - 1, 0)

            fire(0, 0, sem0)
            fire(1, 1, sem1)

            def pair_step(q, rlo):
                g = 2 * q

                def do_even(rl):
                    wait(0, sem0)
                    r = rmw_half(g, 0, rl)
                    fire(g + 2, 0, sem0)
                    return r
                r1 = lax.cond(g < ngr, do_even, lambda rl: rl, rlo)

                def do_odd(rl):
                    wait(1, sem1)
                    r = rmw_half(g + 1, 1, rl)
                    fire(g + 3, 1, sem1)
                    return r
                r2 = lax.cond(g + 1 < ngr, do_odd, lambda rl: rl, r1)
                return r2
            lax.fori_loop(0, lax.shift_right_logical(ngr + 1, 1), pair_step, 0)
            return c

        lax.while_loop(lambda c_: c_ < NCHUNK, round_body, 0)

        zero16 = jnp.zeros((16,), jnp.float32)

        @plsc.parallel_loop(0, RPT, unroll=4)
        def _fix_r(r):
            for v in range(8):
                sl = pl.ds(v * 16, 16)
                a = acc[r, sl]
                acc[r, sl] = jnp.where(a == neg16, zero16, a)
        pltpu.sync_copy(acc.at[pl.ds(0, RPT)], out_hbm.at[pl.ds(lo, RPT)])

    return k(x, packed)


def kernel(x, edge_index0, edge_index1, edge_index2,
           W_fc1, b_fc1, W_proj1, b_proj1,
           W_fc2, b_fc2, W_proj2, b_proj2,
           W_fc3, b_fc3, W_proj3, b_proj3):
    pe0 = (edge_index0[1] << SB) | edge_index0[0]
    pe1 = (edge_index1[1] << SB) | edge_index1[0]
    pe2 = (edge_index2[1] << SB) | edge_index2[0]

    n1 = _seg_max_sc(x, pe0)
    xp = jnp.pad(x, ((0, NPAD - N), (0, 0)))
    h1 = _dense(xp, n1, W_fc1, b_fc1, W_proj1, b_proj1, relu=True)

    n2 = _seg_max_sc(h1, pe1)
    h2 = _dense(h1, n2, W_fc2, b_fc2, W_proj2, b_proj2, relu=False)

    n3 = _seg_max_sc(h2, pe2)
    h3 = _dense(h2, n3, W_fc3, b_fc3, W_proj3, b_proj3, relu=False)
    return h3[:N, :1]
